# baseline (speedup 1.0000x reference)
import functools

import jax
import jax.numpy as jnp
from jax.experimental import pallas as pl
from jax.experimental.pallas import tpu as pltpu

_MARGIN = 0.4
_EPS = 1e-12
_LOG2E = 1.4426950408889634
_C = 128


def _hilo_rowsum(y):
    yh = y.astype(jnp.bfloat16)
    yl = (y - yh.astype(jnp.float32)).astype(jnp.bfloat16)
    ones = jnp.ones((1, y.shape[1]), jnp.bfloat16)
    contract = (((1,), (1,)), ((), ()))
    return (jax.lax.dot_general(ones, yh, contract,
                                preferred_element_type=jnp.float32)
            + jax.lax.dot_general(ones, yl, contract,
                                  preferred_element_type=jnp.float32))


def _sq_kernel(x_ref, xb_ref, lab_ref, sq_ref, sqb_ref, cnt_ref):
    x = x_ref[...]
    sq_ref[...] = _hilo_rowsum(x * x)[None]
    xb = xb_ref[...].astype(jnp.float32)
    sqb_ref[...] = _hilo_rowsum(xb * xb)[None]
    cls = jax.lax.broadcasted_iota(jnp.int32, (1, _C), 1)
    hit = jnp.where(lab_ref[...] == cls, 1.0, 0.0)
    cnt_ref[...] = jnp.sum(hit, axis=0, keepdims=True)[None]


def _dist_tile(fT_ref, xj_ref, sqi_ref, sqj_ref):
    prod = jax.lax.dot_general(
        xj_ref[...], fT_ref[...], (((1,), (0,)), ((), ())),
        preferred_element_type=jnp.float32)
    d2 = jnp.maximum((sqj_ref[...] + sqi_ref[...]) + prod, _EPS)
    return d2 * jax.lax.rsqrt(d2)


def _pair_select(n_i, n_j, r, p, q):
    primary = q < n_j - r * p
    i_sel = jnp.where(primary, p, n_i - 1 - p)
    j_sel = jnp.where(primary, r * p + q, q + r * (n_i - 1) - n_j)
    return i_sel, j_sel


def _make_s_kernel(n_i, n_j, BM, BN):
    def _s_kernel(fT_ref, xj_ref, li_ref, lj_ref, sqi_ref, sqj_ref,
                  srow_ref, scol_ref):
        p = pl.program_id(0)
        q = pl.program_id(1)
        i_sel, j_sel = _pair_select(n_i, n_j, BM // BN, p, q)

        d = _dist_tile(fT_ref, xj_ref, sqi_ref, sqj_ref)
        neg = lj_ref[...] != li_ref[...]
        e = jnp.exp2(_MARGIN * _LOG2E - _LOG2E * d)
        rows = j_sel * BN + jax.lax.broadcasted_iota(jnp.int32, (BN, 1), 0)
        cols = i_sel * BM + jax.lax.broadcasted_iota(jnp.int32, (1, BM), 1)
        e = jnp.where(rows > cols, jnp.where(neg, e, 0.0), 0.0)

        rowpart = jnp.sum(e, axis=0, keepdims=True)[None]
        colpart = jax.lax.dot_general(
            jnp.ones((1, BM), jnp.bfloat16), e.astype(jnp.bfloat16),
            (((1,), (1,)), ((), ())),
            preferred_element_type=jnp.float32)[None]

        @pl.when((q == 0) | (q == n_j - (BM // BN) * p))
        def _():
            srow_ref[...] = rowpart

        @pl.when((q != 0) & (q != n_j - (BM // BN) * p))
        def _():
            srow_ref[...] = srow_ref[...] + rowpart

        @pl.when((p == 0) & (q == 0))
        def _():
            scol_ref[...] = jnp.zeros_like(scol_ref)

        scol_ref[pl.ds(j_sel, 1)] = scol_ref[pl.ds(j_sel, 1)] + colpart

    return _s_kernel


def _make_loss_kernel(n_i, n_j, BM, BN):
    def _loss_kernel(fT_ref, xj_ref, li_ref, lj_ref, sqi_ref, sqj_ref,
                     sqbi_ref, si_ref, sj_ref, loss_ref):
        p = pl.program_id(0)
        q = pl.program_id(1)
        i_sel, j_sel = _pair_select(n_i, n_j, BM // BN, p, q)

        d = _dist_tile(fT_ref, xj_ref, sqi_ref, sqj_ref)
        pos = lj_ref[...] == li_ref[...]
        jv = jnp.log(sj_ref[...] + si_ref[...]) + d
        jv = jnp.where(pos, jnp.maximum(jv, 0.0), 0.0)
        rows = j_sel * BN + jax.lax.broadcasted_iota(jnp.int32, (BN, 1), 0)
        cols = i_sel * BM + jax.lax.broadcasted_iota(jnp.int32, (1, BM), 1)
        jv = jnp.where(rows > cols, jv, 0.0)
        part = jnp.sum(jv * jv, axis=0, keepdims=True)[None]

        @pl.when((q == 0) | (q == n_j - (BM // BN) * p))
        def _():
            x = jnp.maximum(2.0 * sqi_ref[...] - 2.0 * sqbi_ref[...], _EPS)
            d_aa = x * jax.lax.rsqrt(x)
            si = si_ref[...]
            jd = jnp.maximum(jnp.log(si + si) + d_aa, 0.0)
            loss_ref[...] = part + (0.5 * jd * jd)[None]

        @pl.when((q != 0) & (q != n_j - (BM // BN) * p))
        def _():
            loss_ref[...] = loss_ref[...] + part

    return _loss_kernel


@jax.jit
def kernel(features, labels):
    N, D = features.shape
    BM, BN = 2048, 512
    n_i, n_j = N // BM, N // BN

    labels = labels.astype(jnp.int32)
    lab_row = labels.reshape(1, N)
    lab_col = labels.reshape(N, 1)

    f_bf = features.astype(jnp.bfloat16)
    fTn2 = (-2.0 * features).T.astype(jnp.bfloat16)

    BT = 2048
    sq, sqb, cnt = pl.pallas_call(
        _sq_kernel,
        grid=(N // BT,),
        in_specs=[pl.BlockSpec((BT, D), lambda i: (i, 0)),
                  pl.BlockSpec((BT, D), lambda i: (i, 0)),
                  pl.BlockSpec((BT, 1), lambda i: (i, 0))],
        out_specs=[pl.BlockSpec((1, 1, BT), lambda i: (i, 0, 0)),
                   pl.BlockSpec((1, 1, BT), lambda i: (i, 0, 0)),
                   pl.BlockSpec((1, 1, _C), lambda i: (i, 0, 0))],
        out_shape=[jax.ShapeDtypeStruct((N // BT, 1, BT), jnp.float32),
                   jax.ShapeDtypeStruct((N // BT, 1, BT), jnp.float32),
                   jax.ShapeDtypeStruct((N // BT, 1, _C), jnp.float32)],
        compiler_params=pltpu.CompilerParams(
            dimension_semantics=("arbitrary",)),
    )(features, f_bf, lab_col)
    sq_row = sq.reshape(1, N)
    sq_col = sq.reshape(N, 1)
    sqb_row = sqb.reshape(1, N)
    counts = jnp.sum(cnt, axis=0)
    num_pos = jnp.sum(counts * counts)

    acc_shape = jax.ShapeDtypeStruct((n_i, 1, BM), jnp.float32)
    params = pltpu.CompilerParams(
        dimension_semantics=("arbitrary", "arbitrary"))

    def tri_spec(shape, blockfun):
        def index_map(p, q):
            i_sel, j_sel = _pair_select(n_i, n_j, BM // BN, p, q)
            return blockfun(i_sel, j_sel)
        return pl.BlockSpec(shape, index_map)

    tri_grid = (n_i // 2, 2 * n_j - (BM // BN) * (n_i - 1))
    srow, scol = pl.pallas_call(
        _make_s_kernel(n_i, n_j, BM, BN),
        grid=tri_grid,
        in_specs=[tri_spec((D, BM), lambda i, j: (0, i)),
                  tri_spec((BN, D), lambda i, j: (j, 0)),
                  tri_spec((1, BM), lambda i, j: (0, i)),
                  tri_spec((BN, 1), lambda i, j: (j, 0)),
                  tri_spec((1, BM), lambda i, j: (0, i)),
                  tri_spec((BN, 1), lambda i, j: (j, 0))],
        out_specs=[tri_spec((1, 1, BM), lambda i, j: (i, 0, 0)),
                   pl.BlockSpec((n_j, 1, BN), lambda p, q: (0, 0, 0))],
        out_shape=[acc_shape,
                   jax.ShapeDtypeStruct((n_j, 1, BN), jnp.float32)],
        compiler_params=params,
    )(fTn2, f_bf, lab_row, lab_col, sq_row, sq_col)

    s = srow.reshape(N) + scol.reshape(N)
    s_row = s.reshape(1, N)
    s_col = s.reshape(N, 1)

    loss_rows = pl.pallas_call(
        _make_loss_kernel(n_i, n_j, BM, BN),
        grid=tri_grid,
        in_specs=[tri_spec((D, BM), lambda i, j: (0, i)),
                  tri_spec((BN, D), lambda i, j: (j, 0)),
                  tri_spec((1, BM), lambda i, j: (0, i)),
                  tri_spec((BN, 1), lambda i, j: (j, 0)),
                  tri_spec((1, BM), lambda i, j: (0, i)),
                  tri_spec((BN, 1), lambda i, j: (j, 0)),
                  tri_spec((1, BM), lambda i, j: (0, i)),
                  tri_spec((1, BM), lambda i, j: (0, i)),
                  tri_spec((BN, 1), lambda i, j: (j, 0))],
        out_specs=tri_spec((1, 1, BM), lambda i, j: (i, 0, 0)),
        out_shape=acc_shape,
        compiler_params=params,
    )(fTn2, f_bf, lab_row, lab_col, sq_row, sq_col, sqb_row, s_row, s_col)

    return jnp.sum(loss_rows) / num_pos


# speedup vs baseline: 1.1222x; 1.1222x over previous
import functools

import jax
import jax.numpy as jnp
from jax.experimental import pallas as pl
from jax.experimental.pallas import tpu as pltpu

_MARGIN = 0.4
_EPS = 1e-12
_LOG2E = 1.4426950408889634
_C = 128


def _hilo_rowsum(y):
    yh = y.astype(jnp.bfloat16)
    yl = (y - yh.astype(jnp.float32)).astype(jnp.bfloat16)
    ones = jnp.ones((1, y.shape[1]), jnp.bfloat16)
    contract = (((1,), (1,)), ((), ()))
    return (jax.lax.dot_general(ones, yh, contract,
                                preferred_element_type=jnp.float32)
            + jax.lax.dot_general(ones, yl, contract,
                                  preferred_element_type=jnp.float32))


def _sq_kernel(x_ref, xb_ref, lab_ref, sq_ref, sqb_ref, cnt_ref):
    x = x_ref[...]
    sq_ref[...] = _hilo_rowsum(x * x)[None]
    xb = xb_ref[...].astype(jnp.float32)
    sqb_ref[...] = _hilo_rowsum(xb * xb)[None]
    cls = jax.lax.broadcasted_iota(jnp.int32, (1, _C), 1)
    hit = jnp.where(lab_ref[...] == cls, 1.0, 0.0)
    cnt_ref[...] = jnp.sum(hit, axis=0, keepdims=True)[None]


def _dist_tile(fT_ref, xj_ref, sqi_ref, sqj_ref):
    prod = jax.lax.dot_general(
        xj_ref[...], fT_ref[...], (((1,), (0,)), ((), ())),
        preferred_element_type=jnp.float32)
    d2 = jnp.maximum((sqj_ref[...] + sqi_ref[...]) + prod, _EPS)
    return d2 * jax.lax.rsqrt(d2)


def _pair_select(n_i, n_j, r, p, q):
    primary = q < n_j - r * p
    i_sel = jnp.where(primary, p, n_i - 1 - p)
    j_sel = jnp.where(primary, r * p + q, q + r * (n_i - 1) - n_j)
    return i_sel, j_sel


def _make_s_kernel(n_i, n_j, BM, BN):
    def _s_kernel(fT_ref, xj_ref, li_ref, lj_ref, sqi_ref, sqj_ref,
                  srow_ref, scol_ref):
        p = pl.program_id(0)
        q = pl.program_id(1)
        i_sel, j_sel = _pair_select(n_i, n_j, BM // BN, p, q)

        d = _dist_tile(fT_ref, xj_ref, sqi_ref, sqj_ref)
        neg = lj_ref[...] != li_ref[...]
        e = jnp.exp2(_MARGIN * _LOG2E - _LOG2E * d)
        rows = j_sel * BN + jax.lax.broadcasted_iota(jnp.int32, (BN, 1), 0)
        cols = i_sel * BM + jax.lax.broadcasted_iota(jnp.int32, (1, BM), 1)
        e = jnp.where(rows > cols, jnp.where(neg, e, 0.0), 0.0)

        rowpart = jnp.sum(e, axis=0, keepdims=True)[None]
        colpart = jax.lax.dot_general(
            jnp.ones((1, BM), jnp.bfloat16), e.astype(jnp.bfloat16),
            (((1,), (1,)), ((), ())),
            preferred_element_type=jnp.float32)[None]

        @pl.when((q == 0) | (q == n_j - (BM // BN) * p))
        def _():
            srow_ref[...] = rowpart

        @pl.when((q != 0) & (q != n_j - (BM // BN) * p))
        def _():
            srow_ref[...] = srow_ref[...] + rowpart

        @pl.when((p == 0) & (q == 0))
        def _():
            scol_ref[...] = jnp.zeros_like(scol_ref)

        scol_ref[pl.ds(j_sel, 1)] = scol_ref[pl.ds(j_sel, 1)] + colpart

    return _s_kernel


def _make_loss_kernel(n_i, n_j, BM, BN):
    def _loss_kernel(fT_ref, xj_ref, li_ref, lj_ref, sqi_ref, sqj_ref,
                     sqbi_ref, si_ref, sj_ref, loss_ref):
        p = pl.program_id(0)
        q = pl.program_id(1)
        i_sel, j_sel = _pair_select(n_i, n_j, BM // BN, p, q)

        d = _dist_tile(fT_ref, xj_ref, sqi_ref, sqj_ref)
        pos = lj_ref[...] == li_ref[...]
        jv = jnp.log(sj_ref[...] + si_ref[...]) + d
        jv = jnp.where(pos, jnp.maximum(jv, 0.0), 0.0)
        rows = j_sel * BN + jax.lax.broadcasted_iota(jnp.int32, (BN, 1), 0)
        cols = i_sel * BM + jax.lax.broadcasted_iota(jnp.int32, (1, BM), 1)
        jv = jnp.where(rows > cols, jv, 0.0)
        part = jnp.sum(jv * jv, axis=0, keepdims=True)[None]

        @pl.when((q == 0) | (q == n_j - (BM // BN) * p))
        def _():
            x = jnp.maximum(2.0 * sqi_ref[...] - 2.0 * sqbi_ref[...], _EPS)
            d_aa = x * jax.lax.rsqrt(x)
            si = si_ref[...]
            jd = jnp.maximum(jnp.log(si + si) + d_aa, 0.0)
            loss_ref[...] = part + (0.5 * jd * jd)[None]

        @pl.when((q != 0) & (q != n_j - (BM // BN) * p))
        def _():
            loss_ref[...] = loss_ref[...] + part

    return _loss_kernel


@jax.jit
def kernel(features, labels):
    N, D = features.shape
    BM, BN = 1024, 1024
    n_i, n_j = N // BM, N // BN

    labels = labels.astype(jnp.int32)
    lab_row = labels.reshape(1, N)
    lab_col = labels.reshape(N, 1)

    f_bf = features.astype(jnp.bfloat16)
    fTn2 = (-2.0 * features).T.astype(jnp.bfloat16)

    BT = 2048
    sq, sqb, cnt = pl.pallas_call(
        _sq_kernel,
        grid=(N // BT,),
        in_specs=[pl.BlockSpec((BT, D), lambda i: (i, 0)),
                  pl.BlockSpec((BT, D), lambda i: (i, 0)),
                  pl.BlockSpec((BT, 1), lambda i: (i, 0))],
        out_specs=[pl.BlockSpec((1, 1, BT), lambda i: (i, 0, 0)),
                   pl.BlockSpec((1, 1, BT), lambda i: (i, 0, 0)),
                   pl.BlockSpec((1, 1, _C), lambda i: (i, 0, 0))],
        out_shape=[jax.ShapeDtypeStruct((N // BT, 1, BT), jnp.float32),
                   jax.ShapeDtypeStruct((N // BT, 1, BT), jnp.float32),
                   jax.ShapeDtypeStruct((N // BT, 1, _C), jnp.float32)],
        compiler_params=pltpu.CompilerParams(
            dimension_semantics=("arbitrary",)),
    )(features, f_bf, lab_col)
    sq_row = sq.reshape(1, N)
    sq_col = sq.reshape(N, 1)
    sqb_row = sqb.reshape(1, N)
    counts = jnp.sum(cnt, axis=0)
    num_pos = jnp.sum(counts * counts)

    acc_shape = jax.ShapeDtypeStruct((n_i, 1, BM), jnp.float32)
    params = pltpu.CompilerParams(
        dimension_semantics=("arbitrary", "arbitrary"))

    def tri_spec(shape, blockfun):
        def index_map(p, q):
            i_sel, j_sel = _pair_select(n_i, n_j, BM // BN, p, q)
            return blockfun(i_sel, j_sel)
        return pl.BlockSpec(shape, index_map)

    tri_grid = (n_i // 2, 2 * n_j - (BM // BN) * (n_i - 1))
    srow, scol = pl.pallas_call(
        _make_s_kernel(n_i, n_j, BM, BN),
        grid=tri_grid,
        in_specs=[tri_spec((D, BM), lambda i, j: (0, i)),
                  tri_spec((BN, D), lambda i, j: (j, 0)),
                  tri_spec((1, BM), lambda i, j: (0, i)),
                  tri_spec((BN, 1), lambda i, j: (j, 0)),
                  tri_spec((1, BM), lambda i, j: (0, i)),
                  tri_spec((BN, 1), lambda i, j: (j, 0))],
        out_specs=[tri_spec((1, 1, BM), lambda i, j: (i, 0, 0)),
                   pl.BlockSpec((n_j, 1, BN), lambda p, q: (0, 0, 0))],
        out_shape=[acc_shape,
                   jax.ShapeDtypeStruct((n_j, 1, BN), jnp.float32)],
        compiler_params=params,
    )(fTn2, f_bf, lab_row, lab_col, sq_row, sq_col)

    s = srow.reshape(N) + scol.reshape(N)
    s_row = s.reshape(1, N)
    s_col = s.reshape(N, 1)

    loss_rows = pl.pallas_call(
        _make_loss_kernel(n_i, n_j, BM, BN),
        grid=tri_grid,
        in_specs=[tri_spec((D, BM), lambda i, j: (0, i)),
                  tri_spec((BN, D), lambda i, j: (j, 0)),
                  tri_spec((1, BM), lambda i, j: (0, i)),
                  tri_spec((BN, 1), lambda i, j: (j, 0)),
                  tri_spec((1, BM), lambda i, j: (0, i)),
                  tri_spec((BN, 1), lambda i, j: (j, 0)),
                  tri_spec((1, BM), lambda i, j: (0, i)),
                  tri_spec((1, BM), lambda i, j: (0, i)),
                  tri_spec((BN, 1), lambda i, j: (j, 0))],
        out_specs=tri_spec((1, 1, BM), lambda i, j: (i, 0, 0)),
        out_shape=acc_shape,
        compiler_params=params,
    )(fTn2, f_bf, lab_row, lab_col, sq_row, sq_col, sqb_row, s_row, s_col)

    return jnp.sum(loss_rows) / num_pos
